# baseline (speedup 1.0000x reference)
import functools

import jax
import jax.numpy as jnp
from jax.experimental import pallas as pl
from jax.experimental.pallas import tpu as pltpu
from jax.experimental.pallas import tpu_sc as plsc

_TILE = 256
_CHUNK = 512


def _dispatch_kernel(x_ref, wr_ref, br_ref,
                     pos0_ref, pos1_ref, g0_ref, g1_ref, te_ref, tv_ref):
    S, D = x_ref.shape
    E = wr_ref.shape[1]
    T = _TILE
    f32 = jnp.float32
    hi = jax.lax.Precision.HIGHEST

    x = x_ref[...]
    logits = (jnp.dot(x, wr_ref[...], preferred_element_type=f32)
              + br_ref[...][None, :])
    m = jnp.max(logits, axis=-1, keepdims=True)
    ex = jnp.exp(logits - m)
    p = ex / jnp.sum(ex, axis=-1, keepdims=True)

    lane = jax.lax.broadcasted_iota(jnp.int32, (S, E), 1)
    i1 = jnp.argmax(p, axis=-1)[:, None]
    m1 = jnp.max(p, axis=-1, keepdims=True)
    sel1 = lane == i1
    pr = jnp.where(sel1, -1.0, p)
    i2 = jnp.argmax(pr, axis=-1)[:, None]
    m2 = jnp.max(pr, axis=-1, keepdims=True)
    sel2 = lane == i2
    den = m1 + m2
    g0 = m1 / den
    g1 = m2 / den

    M1 = sel1.astype(jnp.bfloat16)
    M2 = sel2.astype(jnp.bfloat16)
    M = (M1 + M2).astype(jnp.bfloat16)

    RC = 512
    L_r = jax.lax.broadcasted_iota(jnp.int32, (RC, RC), 0)
    L_c = jax.lax.broadcasted_iota(jnp.int32, (RC, RC), 1)
    Lstrict = (L_c < L_r).astype(jnp.bfloat16)
    rank_chunks = []
    carry = jnp.zeros((1, E), f32)
    for ci in range(S // RC):
        Mc = M[ci * RC:(ci + 1) * RC, :]
        within = jnp.dot(Lstrict, Mc, preferred_element_type=f32)
        rank_chunks.append(within + carry)
        carry = carry + jnp.sum(Mc.astype(f32), axis=0, keepdims=True)
    rank = jnp.concatenate(rank_chunks, axis=0)
    cnt = carry

    ntiles = jnp.ceil(cnt / T)
    padcnt = ntiles * T
    e_r = jax.lax.broadcasted_iota(jnp.int32, (E, E), 0)
    e_c = jax.lax.broadcasted_iota(jnp.int32, (E, E), 1)
    excl = (e_r < e_c).astype(f32)
    off = jnp.dot(padcnt, excl, preferred_element_type=f32,
                  precision=hi)

    slot = off + rank
    pos0 = jnp.sum(jnp.where(sel1, slot, 0.0), axis=1, keepdims=True)
    pos1 = jnp.sum(jnp.where(sel2, slot, 0.0), axis=1, keepdims=True)
    pos0_ref[...] = pos0.astype(jnp.int32)
    pos1_ref[...] = pos1.astype(jnp.int32)
    g0_ref[...] = g0
    g1_ref[...] = g1

    NT = te_ref.shape[1]
    ti = jax.lax.broadcasted_iota(jnp.int32, (1, NT), 1).astype(f32)
    tstart = off / T
    e_lane = jax.lax.broadcasted_iota(jnp.int32, (1, E), 1)
    te = jnp.zeros((1, NT), f32)
    for e in range(E):
        s_e = jnp.sum(jnp.where(e_lane == e, tstart, 0.0),
                      axis=1, keepdims=True)
        te = te + (ti >= s_e).astype(f32)
    te_ref[...] = (te - 1.0).astype(jnp.int32)
    total_tiles = jnp.sum(ntiles, axis=1, keepdims=True)
    tv_ref[...] = (ti < total_tiles).astype(jnp.int32)


def _add_kernel(a_ref, b_ref, g0_ref, g1_ref, o_ref):
    o_ref[...] = a_ref[...] * g0_ref[...] + b_ref[...] * g1_ref[...]


def _group_mm_kernel(te_ref, tv_ref, xs_ref, w1_ref, b1_ref, w2_ref, b2_ref,
                     ys_ref):
    i = pl.program_id(0)

    @pl.when(tv_ref[i] == 1)
    def _():
        xt = xs_ref[...].astype(jnp.bfloat16)
        h = jnp.dot(xt, w1_ref[0], preferred_element_type=jnp.float32)
        h = jnp.maximum(h + b1_ref[0], 0.0).astype(jnp.bfloat16)
        o = jnp.dot(h, w2_ref[0], preferred_element_type=jnp.float32)
        ys_ref[...] = o + b2_ref[0]


def _make_sc_dispatch(S, PP, D):
    info = plsc.get_sparse_core_info()
    nc, ns, L = info.num_cores, info.num_subcores, info.num_lanes
    nw = nc * ns
    rpw = PP // nw
    half = rpw // 2
    tps = S // ns
    zps = PP // ns
    mesh = plsc.VectorSubcoreMesh(core_axis_name="c", subcore_axis_name="s")

    @functools.partial(
        pl.kernel, mesh=mesh,
        out_type=jax.ShapeDtypeStruct((PP, D), jnp.float32),
        scratch_types=[
            pltpu.VMEM((tps,), jnp.int32),
            pltpu.VMEM((tps,), jnp.int32),
            pltpu.VMEM((tps,), jnp.int32),
            pltpu.VMEM((zps,), jnp.int32),
            pltpu.VMEM((half,), jnp.int32),
            pltpu.VMEM((half,), jnp.int32),
            pltpu.VMEM((half, D), jnp.float32),
            pltpu.VMEM_SHARED((PP,), jnp.int32),
            pltpu.SemaphoreType.DMA,
        ],
    )
    def k(x_hbm, p0_hbm, p1_hbm, xs_hbm,
          p0_v, p1_v, tid_v, z_v, ta_v, tb_v, rows_v, shared, s0):
        c = jax.lax.axis_index("c")
        s = jax.lax.axis_index("s")
        wbase = (s * nc + c) * rpw
        tbase = s * tps

        zero = jnp.zeros((L,), jnp.int32)
        for j in range(zps // L):
            z_v[pl.ds(j * L, L)] = zero
        pltpu.sync_copy(z_v, shared.at[pl.ds(s * zps, zps)])

        pltpu.sync_copy(p0_hbm.at[pl.ds(tbase, tps)], p0_v)
        pltpu.sync_copy(p1_hbm.at[pl.ds(tbase, tps)], p1_v)
        for j in range(tps // L):
            tid_v[pl.ds(j * L, L)] = (
                jax.lax.broadcasted_iota(jnp.int32, (L,), 0) + (tbase + j * L))
        plsc.subcore_barrier()
        pltpu.sync_copy(tid_v, shared.at[p0_v])
        pltpu.sync_copy(tid_v, shared.at[p1_v])
        plsc.subcore_barrier()

        pltpu.sync_copy(shared.at[pl.ds(wbase, half)], ta_v)
        pltpu.sync_copy(shared.at[pl.ds(wbase + half, half)], tb_v)
        pltpu.async_copy(x_hbm.at[ta_v], rows_v, s0).wait()
        pltpu.sync_copy(rows_v, xs_hbm.at[pl.ds(wbase, half)])
        pltpu.async_copy(x_hbm.at[tb_v], rows_v, s0).wait()
        pltpu.sync_copy(rows_v, xs_hbm.at[pl.ds(wbase + half, half)])

    return k


def _make_sc_combine(S, PP, D):
    info = plsc.get_sparse_core_info()
    nw = info.num_cores * info.num_subcores
    tpw = S // nw
    mesh = plsc.VectorSubcoreMesh(core_axis_name="c", subcore_axis_name="s")

    @functools.partial(
        pl.kernel, mesh=mesh,
        out_type=(
            jax.ShapeDtypeStruct((S, D), jnp.float32),
            jax.ShapeDtypeStruct((S, D), jnp.float32),
        ),
        scratch_types=[
            pltpu.VMEM((tpw,), jnp.int32),
            pltpu.VMEM((tpw,), jnp.int32),
            pltpu.VMEM((tpw, D), jnp.float32),
            pltpu.VMEM((tpw, D), jnp.float32),
            pltpu.SemaphoreType.DMA,
            pltpu.SemaphoreType.DMA,
        ],
    )
    def k(ys_hbm, p0_hbm, p1_hbm, ya_hbm, yb_hbm,
          i0_v, i1_v, r0_v, r1_v, s0, s1):
        c = jax.lax.axis_index("c")
        s = jax.lax.axis_index("s")
        base = (s * info.num_cores + c) * tpw
        pltpu.sync_copy(p0_hbm.at[pl.ds(base, tpw)], i0_v)
        pltpu.sync_copy(p1_hbm.at[pl.ds(base, tpw)], i1_v)
        c0 = pltpu.async_copy(ys_hbm.at[i0_v], r0_v, s0)
        c1 = pltpu.async_copy(ys_hbm.at[i1_v], r1_v, s1)
        c0.wait()
        c1.wait()
        pltpu.sync_copy(r0_v, ya_hbm.at[pl.ds(base, tpw)])
        pltpu.sync_copy(r1_v, yb_hbm.at[pl.ds(base, tpw)])

    return k


@jax.jit
def kernel(x, Wr, br, W1, b1, W2, b2):
    B, S, D = x.shape
    E = Wr.shape[1]
    H = W1.shape[2]
    T = _TILE
    MAX_TILES = (2 * S) // T + E
    PP = MAX_TILES * T
    NT = 128

    xs_flat = x.reshape(B * S, D)

    pos0, pos1, g0, g1, te, tv = pl.pallas_call(
        _dispatch_kernel,
        out_shape=(
            jax.ShapeDtypeStruct((B * S, 1), jnp.int32),
            jax.ShapeDtypeStruct((B * S, 1), jnp.int32),
            jax.ShapeDtypeStruct((B * S, 1), jnp.float32),
            jax.ShapeDtypeStruct((B * S, 1), jnp.float32),
            jax.ShapeDtypeStruct((1, NT), jnp.int32),
            jax.ShapeDtypeStruct((1, NT), jnp.int32),
        ),
    )(xs_flat, Wr, br)

    xs = _make_sc_dispatch(B * S, PP, D)(
        xs_flat, pos0.reshape(B * S), pos1.reshape(B * S))

    w1_16 = W1.astype(jnp.bfloat16)
    w2_16 = W2.astype(jnp.bfloat16)

    ys = pl.pallas_call(
        _group_mm_kernel,
        grid_spec=pltpu.PrefetchScalarGridSpec(
            num_scalar_prefetch=2,
            grid=(MAX_TILES,),
            in_specs=[
                pl.BlockSpec((T, D), lambda i, te, tv: (i, 0)),
                pl.BlockSpec((1, D, H), lambda i, te, tv: (te[i], 0, 0)),
                pl.BlockSpec((1, 1, H), lambda i, te, tv: (te[i], 0, 0)),
                pl.BlockSpec((1, H, D), lambda i, te, tv: (te[i], 0, 0)),
                pl.BlockSpec((1, 1, D), lambda i, te, tv: (te[i], 0, 0)),
            ],
            out_specs=pl.BlockSpec((T, D), lambda i, te, tv: (i, 0)),
        ),
        out_shape=jax.ShapeDtypeStruct((PP, D), jnp.float32),
        compiler_params=pltpu.CompilerParams(
            dimension_semantics=("arbitrary",),
        ),
    )(te[0, :MAX_TILES], tv[0, :MAX_TILES], xs,
      w1_16, b1.reshape(E, 1, H), w2_16, b2.reshape(E, 1, D))

    ya, yb = _make_sc_combine(B * S, PP, D)(
        ys, pos0.reshape(B * S), pos1.reshape(B * S))

    NB = 512
    out = pl.pallas_call(
        _add_kernel,
        grid=((B * S) // NB,),
        in_specs=[
            pl.BlockSpec((NB, D), lambda i: (i, 0)),
            pl.BlockSpec((NB, D), lambda i: (i, 0)),
            pl.BlockSpec((NB, 1), lambda i: (i, 0)),
            pl.BlockSpec((NB, 1), lambda i: (i, 0)),
        ],
        out_specs=pl.BlockSpec((NB, D), lambda i: (i, 0)),
        out_shape=jax.ShapeDtypeStruct((B * S, D), jnp.float32),
    )(ya, yb, g0, g1)
    return out.reshape(B, S, D)


# speedup vs baseline: 1.4207x; 1.4207x over previous
import functools

import jax
import jax.numpy as jnp
from jax.experimental import pallas as pl
from jax.experimental.pallas import tpu as pltpu
from jax.experimental.pallas import tpu_sc as plsc

_TILE = 256
_CHUNK = 512


def _dispatch_kernel(x_ref, wr_ref, br_ref,
                     pos0_ref, pos1_ref, g0_ref, g1_ref, te_ref, tv_ref):
    S, D = x_ref.shape
    E = wr_ref.shape[1]
    T = _TILE
    f32 = jnp.float32
    hi = jax.lax.Precision.HIGHEST

    x = x_ref[...]
    logits = (jnp.dot(x, wr_ref[...], preferred_element_type=f32)
              + br_ref[...][None, :])
    m = jnp.max(logits, axis=-1, keepdims=True)
    ex = jnp.exp(logits - m)
    p = ex / jnp.sum(ex, axis=-1, keepdims=True)

    lane = jax.lax.broadcasted_iota(jnp.int32, (S, E), 1)
    i1 = jnp.argmax(p, axis=-1)[:, None]
    m1 = jnp.max(p, axis=-1, keepdims=True)
    sel1 = lane == i1
    pr = jnp.where(sel1, -1.0, p)
    i2 = jnp.argmax(pr, axis=-1)[:, None]
    m2 = jnp.max(pr, axis=-1, keepdims=True)
    sel2 = lane == i2
    den = m1 + m2
    g0 = m1 / den
    g1 = m2 / den

    M1 = sel1.astype(jnp.bfloat16)
    M2 = sel2.astype(jnp.bfloat16)
    M = (M1 + M2).astype(jnp.bfloat16)

    RC = 512
    L_r = jax.lax.broadcasted_iota(jnp.int32, (RC, RC), 0)
    L_c = jax.lax.broadcasted_iota(jnp.int32, (RC, RC), 1)
    Lstrict = (L_c < L_r).astype(jnp.bfloat16)
    rank_chunks = []
    carry = jnp.zeros((1, E), f32)
    for ci in range(S // RC):
        Mc = M[ci * RC:(ci + 1) * RC, :]
        within = jnp.dot(Lstrict, Mc, preferred_element_type=f32)
        rank_chunks.append(within + carry)
        carry = carry + jnp.sum(Mc.astype(f32), axis=0, keepdims=True)
    rank = jnp.concatenate(rank_chunks, axis=0)
    cnt = carry

    ntiles = jnp.ceil(cnt / T)
    padcnt = ntiles * T
    e_r = jax.lax.broadcasted_iota(jnp.int32, (E, E), 0)
    e_c = jax.lax.broadcasted_iota(jnp.int32, (E, E), 1)
    excl = (e_r < e_c).astype(f32)
    off = jnp.dot(padcnt, excl, preferred_element_type=f32,
                  precision=hi)

    slot = off + rank
    pos0 = jnp.sum(jnp.where(sel1, slot, 0.0), axis=1, keepdims=True)
    pos1 = jnp.sum(jnp.where(sel2, slot, 0.0), axis=1, keepdims=True)
    pos0_ref[...] = pos0.astype(jnp.int32)
    pos1_ref[...] = pos1.astype(jnp.int32)
    g0_ref[...] = g0
    g1_ref[...] = g1

    NT = te_ref.shape[1]
    ti = jax.lax.broadcasted_iota(jnp.int32, (1, NT), 1).astype(f32)
    tstart = off / T
    e_lane = jax.lax.broadcasted_iota(jnp.int32, (1, E), 1)
    te = jnp.zeros((1, NT), f32)
    for e in range(E):
        s_e = jnp.sum(jnp.where(e_lane == e, tstart, 0.0),
                      axis=1, keepdims=True)
        te = te + (ti >= s_e).astype(f32)
    te_f = te - 1.0
    te_ref[...] = te_f.astype(jnp.int32)
    offend = off + cnt
    oend = jnp.zeros((1, NT), f32)
    for e in range(E):
        oe = jnp.sum(jnp.where(e_lane == e, offend, 0.0),
                     axis=1, keepdims=True)
        oend = oend + jnp.where(te_f == e, oe, 0.0)
    rc = jnp.clip(oend - ti * T, 0.0, float(T))
    tv_ref[...] = rc.astype(jnp.int32)


def _add_kernel(a_ref, b_ref, g0_ref, g1_ref, o_ref):
    o_ref[...] = a_ref[...] * g0_ref[...] + b_ref[...] * g1_ref[...]


def _fused_mm_combine(te_ref, rc_ref, xs_ref, w1_ref, b1_ref, w2_ref,
                      b2_ref, pos0_ref, pos1_ref, g0_ref, g1_ref,
                      out_ref, ys_scr):
    MT = rc_ref.shape[0]
    T = xs_ref.shape[0]
    D = xs_ref.shape[1]
    S = pos0_ref.shape[0]
    CC = _CHUNK
    i = pl.program_id(0)

    @pl.when(i < MT)
    def _tile():
        rc = rc_ref[i]
        riota = jax.lax.broadcasted_iota(jnp.int32, (T, 1), 0)

        @pl.when(rc > 0)
        def _():
            xt = jnp.where(riota < rc, xs_ref[...], 0.0).astype(jnp.bfloat16)
            h = jnp.dot(xt, w1_ref[0], preferred_element_type=jnp.float32)
            h = jnp.maximum(h + b1_ref[0], 0.0).astype(jnp.bfloat16)
            o = jnp.dot(h, w2_ref[0], preferred_element_type=jnp.float32)
            ys_scr[pl.ds(i * T, T), :] = (o + b2_ref[0]).astype(jnp.bfloat16)

        @pl.when(rc <= 0)
        def _():
            ys_scr[pl.ds(i * T, T), :] = jnp.zeros((T, D), jnp.bfloat16)

    @pl.when(i >= MT)
    def _combine():
        c = i - MT
        base = c * CC
        pcol = jax.lax.broadcasted_iota(jnp.int32, (S, CC), 1) + base
        G = (jnp.where(pcol == pos0_ref[...], g0_ref[...], 0.0)
             + jnp.where(pcol == pos1_ref[...], g1_ref[...], 0.0))
        G16 = G.astype(jnp.bfloat16)
        ysc = ys_scr[pl.ds(base, CC), :]
        acc = jnp.dot(G16, ysc, preferred_element_type=jnp.float32)

        @pl.when(i == MT)
        def _():
            out_ref[...] = acc

        @pl.when(i > MT)
        def _():
            out_ref[...] += acc


def _make_sc_dispatch(S, PP, D):
    info = plsc.get_sparse_core_info()
    nw = info.num_cores * info.num_subcores
    tpw = S // nw
    mesh = plsc.VectorSubcoreMesh(core_axis_name="c", subcore_axis_name="s")

    @functools.partial(
        pl.kernel, mesh=mesh,
        out_type=jax.ShapeDtypeStruct((PP, D), jnp.float32),
        scratch_types=[
            pltpu.VMEM((tpw,), jnp.int32),
            pltpu.VMEM((tpw,), jnp.int32),
            pltpu.VMEM((tpw, D), jnp.float32),
            pltpu.SemaphoreType.DMA,
            pltpu.SemaphoreType.DMA,
        ],
    )
    def k(x_hbm, p0_hbm, p1_hbm, xs_hbm, i0_v, i1_v, rows_v, s0, s1):
        c = jax.lax.axis_index("c")
        s = jax.lax.axis_index("s")
        base = (s * info.num_cores + c) * tpw
        pltpu.sync_copy(p0_hbm.at[pl.ds(base, tpw)], i0_v)
        pltpu.sync_copy(p1_hbm.at[pl.ds(base, tpw)], i1_v)
        pltpu.sync_copy(x_hbm.at[pl.ds(base, tpw)], rows_v)
        c0 = pltpu.async_copy(rows_v, xs_hbm.at[i0_v], s0)
        c1 = pltpu.async_copy(rows_v, xs_hbm.at[i1_v], s1)
        c0.wait()
        c1.wait()

    return k


def _make_sc_combine(S, PP, D):
    info = plsc.get_sparse_core_info()
    nw = info.num_cores * info.num_subcores
    tpw = S // nw
    mesh = plsc.VectorSubcoreMesh(core_axis_name="c", subcore_axis_name="s")

    @functools.partial(
        pl.kernel, mesh=mesh,
        out_type=(
            jax.ShapeDtypeStruct((S, D), jnp.float32),
            jax.ShapeDtypeStruct((S, D), jnp.float32),
        ),
        scratch_types=[
            pltpu.VMEM((tpw,), jnp.int32),
            pltpu.VMEM((tpw,), jnp.int32),
            pltpu.VMEM((tpw, D), jnp.float32),
            pltpu.VMEM((tpw, D), jnp.float32),
            pltpu.SemaphoreType.DMA,
            pltpu.SemaphoreType.DMA,
        ],
    )
    def k(ys_hbm, p0_hbm, p1_hbm, ya_hbm, yb_hbm,
          i0_v, i1_v, r0_v, r1_v, s0, s1):
        c = jax.lax.axis_index("c")
        s = jax.lax.axis_index("s")
        base = (s * info.num_cores + c) * tpw
        pltpu.sync_copy(p0_hbm.at[pl.ds(base, tpw)], i0_v)
        pltpu.sync_copy(p1_hbm.at[pl.ds(base, tpw)], i1_v)
        c0 = pltpu.async_copy(ys_hbm.at[i0_v], r0_v, s0)
        c1 = pltpu.async_copy(ys_hbm.at[i1_v], r1_v, s1)
        c0.wait()
        c1.wait()
        pltpu.sync_copy(r0_v, ya_hbm.at[pl.ds(base, tpw)])
        pltpu.sync_copy(r1_v, yb_hbm.at[pl.ds(base, tpw)])

    return k


@jax.jit
def kernel(x, Wr, br, W1, b1, W2, b2):
    B, S, D = x.shape
    E = Wr.shape[1]
    H = W1.shape[2]
    T = _TILE
    MAX_TILES = (2 * S) // T + E
    PP = MAX_TILES * T
    NT = 128

    xs_flat = x.reshape(B * S, D)

    pos0, pos1, g0, g1, te, tv = pl.pallas_call(
        _dispatch_kernel,
        out_shape=(
            jax.ShapeDtypeStruct((B * S, 1), jnp.int32),
            jax.ShapeDtypeStruct((B * S, 1), jnp.int32),
            jax.ShapeDtypeStruct((B * S, 1), jnp.float32),
            jax.ShapeDtypeStruct((B * S, 1), jnp.float32),
            jax.ShapeDtypeStruct((1, NT), jnp.int32),
            jax.ShapeDtypeStruct((1, NT), jnp.int32),
        ),
    )(xs_flat, Wr, br)

    xs = _make_sc_dispatch(B * S, PP, D)(
        xs_flat, pos0.reshape(B * S), pos1.reshape(B * S))

    w1_16 = W1.astype(jnp.bfloat16)
    w2_16 = W2.astype(jnp.bfloat16)
    NCC = PP // _CHUNK
    MTm1 = MAX_TILES - 1

    out = pl.pallas_call(
        _fused_mm_combine,
        grid_spec=pltpu.PrefetchScalarGridSpec(
            num_scalar_prefetch=2,
            grid=(MAX_TILES + NCC,),
            in_specs=[
                pl.BlockSpec(
                    (T, D),
                    lambda i, te, rc: (jnp.minimum(i, MTm1), 0)),
                pl.BlockSpec(
                    (1, D, H),
                    lambda i, te, rc: (te[jnp.minimum(i, MTm1)], 0, 0)),
                pl.BlockSpec(
                    (1, 1, H),
                    lambda i, te, rc: (te[jnp.minimum(i, MTm1)], 0, 0)),
                pl.BlockSpec(
                    (1, H, D),
                    lambda i, te, rc: (te[jnp.minimum(i, MTm1)], 0, 0)),
                pl.BlockSpec(
                    (1, 1, D),
                    lambda i, te, rc: (te[jnp.minimum(i, MTm1)], 0, 0)),
                pl.BlockSpec((B * S, 1), lambda i, te, rc: (0, 0)),
                pl.BlockSpec((B * S, 1), lambda i, te, rc: (0, 0)),
                pl.BlockSpec((B * S, 1), lambda i, te, rc: (0, 0)),
                pl.BlockSpec((B * S, 1), lambda i, te, rc: (0, 0)),
            ],
            out_specs=pl.BlockSpec((B * S, D), lambda i, te, rc: (0, 0)),
            scratch_shapes=[pltpu.VMEM((PP, D), jnp.bfloat16)],
        ),
        out_shape=jax.ShapeDtypeStruct((B * S, D), jnp.float32),
        compiler_params=pltpu.CompilerParams(
            dimension_semantics=("arbitrary",),
        ),
    )(te[0, :MAX_TILES], tv[0, :MAX_TILES], xs,
      w1_16, b1.reshape(E, 1, H), w2_16, b2.reshape(E, 1, D),
      pos0, pos1, g0, g1)
    return out.reshape(B, S, D)


# speedup vs baseline: 1.5368x; 1.0817x over previous
import functools

import jax
import jax.numpy as jnp
from jax.experimental import pallas as pl
from jax.experimental.pallas import tpu as pltpu
from jax.experimental.pallas import tpu_sc as plsc

_TILE = 256
_CHUNK = 512


def _dispatch_kernel(x_ref, wr_ref, br_ref,
                     pos0_ref, pos1_ref, g0_ref, g1_ref, te_ref, tv_ref):
    S, D = x_ref.shape
    E = wr_ref.shape[1]
    T = _TILE
    f32 = jnp.float32
    hi = jax.lax.Precision.HIGHEST

    x = x_ref[...]
    logits = (jnp.dot(x, wr_ref[...], preferred_element_type=f32)
              + br_ref[...][None, :])
    m = jnp.max(logits, axis=-1, keepdims=True)
    ex = jnp.exp(logits - m)
    p = ex / jnp.sum(ex, axis=-1, keepdims=True)

    lane = jax.lax.broadcasted_iota(jnp.int32, (S, E), 1)
    i1 = jnp.argmax(p, axis=-1)[:, None]
    m1 = jnp.max(p, axis=-1, keepdims=True)
    sel1 = lane == i1
    pr = jnp.where(sel1, -1.0, p)
    i2 = jnp.argmax(pr, axis=-1)[:, None]
    m2 = jnp.max(pr, axis=-1, keepdims=True)
    sel2 = lane == i2
    den = m1 + m2
    g0 = m1 / den
    g1 = m2 / den

    M1 = sel1.astype(jnp.bfloat16)
    M2 = sel2.astype(jnp.bfloat16)
    M = (M1 + M2).astype(jnp.bfloat16)

    RC = 512
    L_r = jax.lax.broadcasted_iota(jnp.int32, (RC, RC), 0)
    L_c = jax.lax.broadcasted_iota(jnp.int32, (RC, RC), 1)
    Lstrict = (L_c < L_r).astype(jnp.bfloat16)
    rank_chunks = []
    carry = jnp.zeros((1, E), f32)
    for ci in range(S // RC):
        Mc = M[ci * RC:(ci + 1) * RC, :]
        within = jnp.dot(Lstrict, Mc, preferred_element_type=f32)
        rank_chunks.append(within + carry)
        carry = carry + jnp.sum(Mc.astype(f32), axis=0, keepdims=True)
    rank = jnp.concatenate(rank_chunks, axis=0)
    cnt = carry

    ntiles = jnp.ceil(cnt / T)
    padcnt = ntiles * T
    e_r = jax.lax.broadcasted_iota(jnp.int32, (E, E), 0)
    e_c = jax.lax.broadcasted_iota(jnp.int32, (E, E), 1)
    excl = (e_r < e_c).astype(f32)
    off = jnp.dot(padcnt, excl, preferred_element_type=f32,
                  precision=hi)

    slot = off + rank
    pos0 = jnp.sum(jnp.where(sel1, slot, 0.0), axis=1, keepdims=True)
    pos1 = jnp.sum(jnp.where(sel2, slot, 0.0), axis=1, keepdims=True)
    pos0_ref[...] = pos0.astype(jnp.int32)
    pos1_ref[...] = pos1.astype(jnp.int32)
    g0_ref[...] = g0
    g1_ref[...] = g1

    NT = te_ref.shape[1]
    ti = jax.lax.broadcasted_iota(jnp.int32, (1, NT), 1).astype(f32)
    tstart = off / T
    e_lane = jax.lax.broadcasted_iota(jnp.int32, (1, E), 1)
    te = jnp.zeros((1, NT), f32)
    for e in range(E):
        s_e = jnp.sum(jnp.where(e_lane == e, tstart, 0.0),
                      axis=1, keepdims=True)
        te = te + (ti >= s_e).astype(f32)
    te_ref[...] = (te - 1.0).astype(jnp.int32)
    total_tiles = jnp.sum(ntiles, axis=1, keepdims=True)
    tv_ref[...] = (ti < total_tiles).astype(jnp.int32)


def _add_kernel(a_ref, b_ref, g0_ref, g1_ref, o_ref):
    o_ref[...] = a_ref[...] * g0_ref[...] + b_ref[...] * g1_ref[...]


def _group_mm_kernel(te_ref, tv_ref, xs_ref, w1_ref, b1_ref, w2_ref, b2_ref,
                     ys_ref):
    i = pl.program_id(0)

    @pl.when(tv_ref[i] == 1)
    def _():
        xt = xs_ref[...].astype(jnp.bfloat16)
        h = jnp.dot(xt, w1_ref[0], preferred_element_type=jnp.float32)
        h = jnp.maximum(h + b1_ref[0], 0.0).astype(jnp.bfloat16)
        o = jnp.dot(h, w2_ref[0], preferred_element_type=jnp.float32)
        ys_ref[...] = o + b2_ref[0]


def _make_sc_dispatch(S, PP, D):
    info = plsc.get_sparse_core_info()
    nw = info.num_cores * info.num_subcores
    tpw = S // nw
    mesh = plsc.VectorSubcoreMesh(core_axis_name="c", subcore_axis_name="s")

    @functools.partial(
        pl.kernel, mesh=mesh,
        out_type=jax.ShapeDtypeStruct((PP, D), jnp.float32),
        scratch_types=[
            pltpu.VMEM((tpw,), jnp.int32),
            pltpu.VMEM((tpw,), jnp.int32),
            pltpu.VMEM((tpw, D), jnp.float32),
            pltpu.SemaphoreType.DMA,
            pltpu.SemaphoreType.DMA,
        ],
    )
    def k(x_hbm, p0_hbm, p1_hbm, xs_hbm, i0_v, i1_v, rows_v, s0, s1):
        c = jax.lax.axis_index("c")
        s = jax.lax.axis_index("s")
        base = (s * info.num_cores + c) * tpw
        pltpu.sync_copy(p0_hbm.at[pl.ds(base, tpw)], i0_v)
        pltpu.sync_copy(p1_hbm.at[pl.ds(base, tpw)], i1_v)
        pltpu.sync_copy(x_hbm.at[pl.ds(base, tpw)], rows_v)
        c0 = pltpu.async_copy(rows_v, xs_hbm.at[i0_v], s0)
        c1 = pltpu.async_copy(rows_v, xs_hbm.at[i1_v], s1)
        c0.wait()
        c1.wait()

    return k


def _make_sc_combine(S, PP, D):
    info = plsc.get_sparse_core_info()
    nw = info.num_cores * info.num_subcores
    tpw = S // nw
    mesh = plsc.VectorSubcoreMesh(core_axis_name="c", subcore_axis_name="s")

    @functools.partial(
        pl.kernel, mesh=mesh,
        out_type=(
            jax.ShapeDtypeStruct((S, D), jnp.float32),
            jax.ShapeDtypeStruct((S, D), jnp.float32),
        ),
        scratch_types=[
            pltpu.VMEM((tpw,), jnp.int32),
            pltpu.VMEM((tpw,), jnp.int32),
            pltpu.VMEM((tpw, D), jnp.float32),
            pltpu.VMEM((tpw, D), jnp.float32),
            pltpu.SemaphoreType.DMA,
            pltpu.SemaphoreType.DMA,
        ],
    )
    def k(ys_hbm, p0_hbm, p1_hbm, ya_hbm, yb_hbm,
          i0_v, i1_v, r0_v, r1_v, s0, s1):
        c = jax.lax.axis_index("c")
        s = jax.lax.axis_index("s")
        base = (s * info.num_cores + c) * tpw
        pltpu.sync_copy(p0_hbm.at[pl.ds(base, tpw)], i0_v)
        pltpu.sync_copy(p1_hbm.at[pl.ds(base, tpw)], i1_v)
        c0 = pltpu.async_copy(ys_hbm.at[i0_v], r0_v, s0)
        c1 = pltpu.async_copy(ys_hbm.at[i1_v], r1_v, s1)
        c0.wait()
        c1.wait()
        pltpu.sync_copy(r0_v, ya_hbm.at[pl.ds(base, tpw)])
        pltpu.sync_copy(r1_v, yb_hbm.at[pl.ds(base, tpw)])

    return k


@jax.jit
def kernel(x, Wr, br, W1, b1, W2, b2):
    B, S, D = x.shape
    E = Wr.shape[1]
    H = W1.shape[2]
    T = _TILE
    MAX_TILES = (2 * S) // T + E
    PP = MAX_TILES * T
    NT = 128

    xs_flat = x.reshape(B * S, D)

    pos0, pos1, g0, g1, te, tv = pl.pallas_call(
        _dispatch_kernel,
        out_shape=(
            jax.ShapeDtypeStruct((B * S, 1), jnp.int32),
            jax.ShapeDtypeStruct((B * S, 1), jnp.int32),
            jax.ShapeDtypeStruct((B * S, 1), jnp.float32),
            jax.ShapeDtypeStruct((B * S, 1), jnp.float32),
            jax.ShapeDtypeStruct((1, NT), jnp.int32),
            jax.ShapeDtypeStruct((1, NT), jnp.int32),
        ),
    )(xs_flat, Wr, br)

    xs = _make_sc_dispatch(B * S, PP, D)(
        xs_flat, pos0.reshape(B * S), pos1.reshape(B * S))

    w1_16 = W1.astype(jnp.bfloat16)
    w2_16 = W2.astype(jnp.bfloat16)

    ys = pl.pallas_call(
        _group_mm_kernel,
        grid_spec=pltpu.PrefetchScalarGridSpec(
            num_scalar_prefetch=2,
            grid=(MAX_TILES,),
            in_specs=[
                pl.BlockSpec((T, D), lambda i, te, tv: (i, 0)),
                pl.BlockSpec((1, D, H), lambda i, te, tv: (te[i], 0, 0)),
                pl.BlockSpec((1, 1, H), lambda i, te, tv: (te[i], 0, 0)),
                pl.BlockSpec((1, H, D), lambda i, te, tv: (te[i], 0, 0)),
                pl.BlockSpec((1, 1, D), lambda i, te, tv: (te[i], 0, 0)),
            ],
            out_specs=pl.BlockSpec((T, D), lambda i, te, tv: (i, 0)),
        ),
        out_shape=jax.ShapeDtypeStruct((PP, D), jnp.float32),
        compiler_params=pltpu.CompilerParams(
            dimension_semantics=("arbitrary",),
        ),
    )(te[0, :MAX_TILES], tv[0, :MAX_TILES], xs,
      w1_16, b1.reshape(E, 1, H), w2_16, b2.reshape(E, 1, D))

    ya, yb = _make_sc_combine(B * S, PP, D)(
        ys, pos0.reshape(B * S), pos1.reshape(B * S))

    NB = 512
    out = pl.pallas_call(
        _add_kernel,
        grid=((B * S) // NB,),
        in_specs=[
            pl.BlockSpec((NB, D), lambda i: (i, 0)),
            pl.BlockSpec((NB, D), lambda i: (i, 0)),
            pl.BlockSpec((NB, 1), lambda i: (i, 0)),
            pl.BlockSpec((NB, 1), lambda i: (i, 0)),
        ],
        out_specs=pl.BlockSpec((NB, D), lambda i: (i, 0)),
        out_shape=jax.ShapeDtypeStruct((B * S, D), jnp.float32),
    )(ya, yb, g0, g1)
    return out.reshape(B, S, D)
